# Initial kernel scaffold; baseline (speedup 1.0000x reference)
import jax
import jax.numpy as jnp
from jax.experimental import pallas as pl


def kernel(x, centers, sigmas, out_centers, output_scaling, output_bias, input_rules, output_rules, var_of_mem):
    raise NotImplementedError("write your pallas kernel here")


# speedup vs baseline: 2.0160x; 2.0160x over previous
import jax
import jax.numpy as jnp
from jax.experimental import pallas as pl

N_VARS = 5
TOTAL_MEM = 24
N_OUT = 2
BB = 512


def _anfis_block(x_ref, c_ref, s_ref, oc_ref, scale_ref, bias_ref,
                 rules_ref, orules_ref, vm_ref, out_ref):
    rpad = rules_ref.shape[1]
    n_oc = oc_ref.shape[1]
    vm = vm_ref[0, :]
    V = (jax.lax.broadcasted_iota(jnp.int32, (N_VARS, TOTAL_MEM), 0)
         == vm[None, :]).astype(jnp.float32)
    xv = jax.lax.dot(x_ref[:, :], V, precision=jax.lax.Precision.HIGHEST)
    c = c_ref[0, :]
    inv = 0.5 / (s_ref[0, :] ** 2)
    fuzz = jnp.exp(-((xv - c[None, :]) ** 2) * inv[None, :])

    minw = None
    for v in range(N_VARS):
        idx = rules_ref[v, :]
        oh = (jax.lax.broadcasted_iota(jnp.int32, (TOTAL_MEM, rpad), 0)
              == idx[None, :]).astype(jnp.float32)
        g = jax.lax.dot(fuzz, oh, precision=jax.lax.Precision.HIGHEST)
        minw = g if minw is None else jnp.minimum(minw, g)
    total = jnp.sum(minw, axis=1, keepdims=True)

    oc = oc_ref[0, :]
    ow_cols = []
    for j in range(N_OUT):
        orj = orules_ref[j, :]
        ohj = (orj[:, None] == jax.lax.broadcasted_iota(
            jnp.int32, (rpad, n_oc), 1)).astype(jnp.float32)
        ow_cols.append(jnp.sum(ohj * oc[None, :], axis=1, keepdims=True))
    ow = jnp.concatenate(ow_cols, axis=1)

    acc = jax.lax.dot(minw, ow, precision=jax.lax.Precision.HIGHEST)
    res = jnp.tanh(acc / jnp.maximum(total, 1e-12))
    out_ref[:, :] = res * scale_ref[0, :][None, :] + bias_ref[0, :][None, :]


def kernel(x, centers, sigmas, out_centers, output_scaling, output_bias,
           input_rules, output_rules, var_of_mem):
    b, nv = x.shape
    r = input_rules.shape[0]
    rpad = ((r + 127) // 128) * 128
    rules_t = jnp.full((8, rpad), TOTAL_MEM + 7, jnp.int32)
    rules_t = rules_t.at[:N_VARS, :r].set(input_rules.T)
    orules_t = jnp.zeros((8, rpad), jnp.int32)
    orules_t = orules_t.at[:N_OUT, :r].set(output_rules.T)
    c2 = centers.reshape(1, -1)
    s2 = sigmas.reshape(1, -1)
    oc2 = out_centers.reshape(1, -1)
    vm2 = var_of_mem.reshape(1, -1)
    sc2 = output_scaling.reshape(1, N_OUT)
    bi2 = output_bias.reshape(1, N_OUT)

    full = lambda shape: pl.BlockSpec(shape, lambda i: (0, 0))
    out = pl.pallas_call(
        _anfis_block,
        grid=(b // BB,),
        in_specs=[
            pl.BlockSpec((BB, nv), lambda i: (i, 0)),
            full((1, TOTAL_MEM)),
            full((1, TOTAL_MEM)),
            full((1, oc2.shape[1])),
            full((1, N_OUT)),
            full((1, N_OUT)),
            full((8, rpad)),
            full((8, rpad)),
            full((1, TOTAL_MEM)),
        ],
        out_specs=pl.BlockSpec((BB, N_OUT), lambda i: (i, 0)),
        out_shape=jax.ShapeDtypeStruct((b, N_OUT), jnp.float32),
    )(x, c2, s2, oc2, sc2, bi2, rules_t, orules_t, vm2)
    return out


# speedup vs baseline: 2.3614x; 1.1713x over previous
import jax
import jax.numpy as jnp
from jax.experimental import pallas as pl

NUM_INPUTS = (2, 7, 5, 5, 5)
STARTS = (0, 2, 9, 14, 19)
N_VARS = 5
TOTAL_MEM = 24
NUM_OC = 18
N_OUT = 2
BB = 512


def _anfis_block(x_ref, c_ref, s_ref, oc_ref, scale_ref, bias_ref,
                 rules_ref, orules_ref, vm_ref, out_ref):
    rpad = rules_ref.shape[1]
    vm = vm_ref[0, :]
    xv = jnp.zeros((x_ref.shape[0], TOTAL_MEM), jnp.float32)
    for v in range(N_VARS):
        xv = jnp.where((vm == v)[None, :], x_ref[:, v][:, None], xv)
    c = c_ref[0, :]
    inv = 0.5 / (s_ref[0, :] ** 2)
    fuzz = jnp.exp(-((xv - c[None, :]) ** 2) * inv[None, :])

    minw = None
    for v in range(N_VARS):
        idx = rules_ref[v, :]
        gv = jnp.zeros((x_ref.shape[0], rpad), jnp.float32)
        for k in range(NUM_INPUTS[v]):
            col = STARTS[v] + k
            gv = jnp.where((idx == col)[None, :], fuzz[:, col][:, None], gv)
        minw = gv if minw is None else jnp.minimum(minw, gv)

    oc = oc_ref[0, :]
    ows = []
    for j in range(N_OUT):
        orj = orules_ref[j, :]
        owj = jnp.zeros((1, rpad), jnp.float32)
        for k in range(NUM_OC):
            owj = jnp.where((orj == k)[None, :], oc[k], owj)
        ows.append(owj)

    total = jnp.sum(minw, axis=1, keepdims=True)
    acc0 = jnp.sum(minw * ows[0], axis=1, keepdims=True)
    acc1 = jnp.sum(minw * ows[1], axis=1, keepdims=True)
    acc = jnp.concatenate([acc0, acc1], axis=1)
    res = jnp.tanh(acc / jnp.maximum(total, 1e-12))
    out_ref[:, :] = res * scale_ref[0, :][None, :] + bias_ref[0, :][None, :]


def kernel(x, centers, sigmas, out_centers, output_scaling, output_bias,
           input_rules, output_rules, var_of_mem):
    b, nv = x.shape
    r = input_rules.shape[0]
    rpad = ((r + 127) // 128) * 128
    rules_t = jnp.full((8, rpad), TOTAL_MEM + 7, jnp.int32)
    rules_t = rules_t.at[:N_VARS, :r].set(input_rules.T)
    orules_t = jnp.full((8, rpad), NUM_OC + 7, jnp.int32)
    orules_t = orules_t.at[:N_OUT, :r].set(output_rules.T)
    c2 = centers.reshape(1, -1)
    s2 = sigmas.reshape(1, -1)
    oc2 = out_centers.reshape(1, -1)
    vm2 = var_of_mem.reshape(1, -1)
    sc2 = output_scaling.reshape(1, N_OUT)
    bi2 = output_bias.reshape(1, N_OUT)

    full = lambda shape: pl.BlockSpec(shape, lambda i: (0, 0))
    out = pl.pallas_call(
        _anfis_block,
        grid=(b // BB,),
        in_specs=[
            pl.BlockSpec((BB, nv), lambda i: (i, 0)),
            full((1, TOTAL_MEM)),
            full((1, TOTAL_MEM)),
            full((1, oc2.shape[1])),
            full((1, N_OUT)),
            full((1, N_OUT)),
            full((8, rpad)),
            full((8, rpad)),
            full((1, TOTAL_MEM)),
        ],
        out_specs=pl.BlockSpec((BB, N_OUT), lambda i: (i, 0)),
        out_shape=jax.ShapeDtypeStruct((b, N_OUT), jnp.float32),
    )(x, c2, s2, oc2, sc2, bi2, rules_t, orules_t, vm2)
    return out


# speedup vs baseline: 5.0105x; 2.1218x over previous
import jax
import jax.numpy as jnp
from jax.experimental import pallas as pl

N_VARS = 5
TOTAL_MEM = 24
NUM_OC = 18
N_OUT = 2
BB = 1024


def _anfis_block(x_ref, c_ref, s_ref, oc_ref, scale_ref, bias_ref,
                 rules_ref, orules_ref, vm_ref, out_ref):
    rpad = rules_ref.shape[1]
    bb = x_ref.shape[0]
    vm = vm_ref[0, :]
    xv = jnp.zeros((bb, TOTAL_MEM), jnp.float32)
    for v in range(N_VARS):
        xv = jnp.where((vm == v)[None, :], x_ref[:, v][:, None], xv)
    c = c_ref[0, :]
    inv = 0.5 / (s_ref[0, :] ** 2)
    fuzz = jnp.exp(-((xv - c[None, :]) ** 2) * inv[None, :])
    fhi = fuzz.astype(jnp.bfloat16)
    flo = (fuzz - fhi.astype(jnp.float32)).astype(jnp.bfloat16)

    minw = None
    for v in range(N_VARS):
        idx = rules_ref[v, :]
        oh = (jax.lax.broadcasted_iota(jnp.int32, (TOTAL_MEM, rpad), 0)
              == idx[None, :]).astype(jnp.bfloat16)
        g = (jax.lax.dot(fhi, oh, preferred_element_type=jnp.float32)
             + jax.lax.dot(flo, oh, preferred_element_type=jnp.float32))
        minw = g if minw is None else jnp.minimum(minw, g)

    oc = oc_ref[0, :]
    ows = []
    for j in range(N_OUT):
        orj = orules_ref[j, :]
        owj = jnp.zeros((1, rpad), jnp.float32)
        for k in range(NUM_OC):
            owj = jnp.where((orj == k)[None, :], oc[k], owj)
        ows.append(owj)

    p0 = minw * ows[0]
    p1 = minw * ows[1]
    a0 = jnp.zeros((bb, 128), jnp.float32)
    a1 = jnp.zeros((bb, 128), jnp.float32)
    at = jnp.zeros((bb, 128), jnp.float32)
    for kk in range(rpad // 128):
        sl = slice(kk * 128, (kk + 1) * 128)
        a0 = a0 + p0[:, sl]
        a1 = a1 + p1[:, sl]
        at = at + minw[:, sl]
    acc0 = jnp.sum(a0, axis=1, keepdims=True)
    acc1 = jnp.sum(a1, axis=1, keepdims=True)
    total = jnp.sum(at, axis=1, keepdims=True)
    acc = jnp.concatenate([acc0, acc1], axis=1)
    res = jnp.tanh(acc / jnp.maximum(total, 1e-12))
    out_ref[:, :] = res * scale_ref[0, :][None, :] + bias_ref[0, :][None, :]


def kernel(x, centers, sigmas, out_centers, output_scaling, output_bias,
           input_rules, output_rules, var_of_mem):
    b, nv = x.shape
    r = input_rules.shape[0]
    rpad = ((r + 127) // 128) * 128
    rules_t = jnp.full((8, rpad), TOTAL_MEM + 7, jnp.int32)
    rules_t = rules_t.at[:N_VARS, :r].set(input_rules.T)
    orules_t = jnp.full((8, rpad), NUM_OC + 7, jnp.int32)
    orules_t = orules_t.at[:N_OUT, :r].set(output_rules.T)
    c2 = centers.reshape(1, -1)
    s2 = sigmas.reshape(1, -1)
    oc2 = out_centers.reshape(1, -1)
    vm2 = var_of_mem.reshape(1, -1)
    sc2 = output_scaling.reshape(1, N_OUT)
    bi2 = output_bias.reshape(1, N_OUT)

    full = lambda shape: pl.BlockSpec(shape, lambda i: (0, 0))
    out = pl.pallas_call(
        _anfis_block,
        grid=(b // BB,),
        in_specs=[
            pl.BlockSpec((BB, nv), lambda i: (i, 0)),
            full((1, TOTAL_MEM)),
            full((1, TOTAL_MEM)),
            full((1, oc2.shape[1])),
            full((1, N_OUT)),
            full((1, N_OUT)),
            full((8, rpad)),
            full((8, rpad)),
            full((1, TOTAL_MEM)),
        ],
        out_specs=pl.BlockSpec((BB, N_OUT), lambda i: (i, 0)),
        out_shape=jax.ShapeDtypeStruct((b, N_OUT), jnp.float32),
    )(x, c2, s2, oc2, sc2, bi2, rules_t, orules_t, vm2)
    return out


# speedup vs baseline: 6.7631x; 1.3498x over previous
import jax
import jax.numpy as jnp
from jax.experimental import pallas as pl

N_VARS = 5
TOTAL_MEM = 24
NUM_OC = 18
N_OUT = 2
BB = 512


def _anfis_block(x_ref, c_ref, s_ref, oc_ref, scale_ref, bias_ref,
                 rules_ref, orules_ref, vm_ref, out_ref):
    rpad = rules_ref.shape[1] // N_VARS
    bb = x_ref.shape[0]
    vm = vm_ref[0, :]
    xv = jnp.zeros((bb, TOTAL_MEM), jnp.float32)
    for v in range(N_VARS):
        xv = jnp.where((vm == v)[None, :], x_ref[:, v][:, None], xv)
    c = c_ref[0, :]
    inv = 0.5 / (s_ref[0, :] ** 2)
    fuzz = jnp.exp(-((xv - c[None, :]) ** 2) * inv[None, :])
    fhi = fuzz.astype(jnp.bfloat16)
    flo = (fuzz - fhi.astype(jnp.float32)).astype(jnp.bfloat16)
    lhs = jnp.concatenate([fhi, flo], axis=1)

    idx = rules_ref[0, :]
    oh = (jax.lax.broadcasted_iota(jnp.int32, (TOTAL_MEM, N_VARS * rpad), 0)
          == idx[None, :]).astype(jnp.bfloat16)
    oh2 = jnp.concatenate([oh, oh], axis=0)
    G = jax.lax.dot(lhs, oh2, preferred_element_type=jnp.float32)

    oc = oc_ref[0, :]
    ows = []
    for j in range(N_OUT):
        orj = orules_ref[j, :]
        owj = jnp.zeros((1, rpad), jnp.float32)
        for k in range(NUM_OC):
            owj = jnp.where((orj == k)[None, :], oc[k], owj)
        ows.append(owj)

    a0 = jnp.zeros((bb, 128), jnp.float32)
    a1 = jnp.zeros((bb, 128), jnp.float32)
    at = jnp.zeros((bb, 128), jnp.float32)
    for kk in range(rpad // 128):
        base = kk * 128
        m = G[:, base:base + 128]
        for v in range(1, N_VARS):
            m = jnp.minimum(m, G[:, v * rpad + base:v * rpad + base + 128])
        a0 = a0 + m * ows[0][:, base:base + 128]
        a1 = a1 + m * ows[1][:, base:base + 128]
        at = at + m
    acc0 = jnp.sum(a0, axis=1, keepdims=True)
    acc1 = jnp.sum(a1, axis=1, keepdims=True)
    total = jnp.sum(at, axis=1, keepdims=True)
    acc = jnp.concatenate([acc0, acc1], axis=1)
    res = jnp.tanh(acc / jnp.maximum(total, 1e-12))
    out_ref[:, :] = res * scale_ref[0, :][None, :] + bias_ref[0, :][None, :]


def kernel(x, centers, sigmas, out_centers, output_scaling, output_bias,
           input_rules, output_rules, var_of_mem):
    b, nv = x.shape
    r = input_rules.shape[0]
    rpad = ((r + 127) // 128) * 128
    rules_cat = jnp.full((N_VARS, rpad), TOTAL_MEM + 7, jnp.int32)
    rules_cat = rules_cat.at[:, :r].set(input_rules.T)
    rules_cat = jnp.broadcast_to(rules_cat.reshape(1, N_VARS * rpad),
                                 (8, N_VARS * rpad))
    orules_t = jnp.full((8, rpad), NUM_OC + 7, jnp.int32)
    orules_t = orules_t.at[:N_OUT, :r].set(output_rules.T)
    c2 = centers.reshape(1, -1)
    s2 = sigmas.reshape(1, -1)
    oc2 = out_centers.reshape(1, -1)
    vm2 = var_of_mem.reshape(1, -1)
    sc2 = output_scaling.reshape(1, N_OUT)
    bi2 = output_bias.reshape(1, N_OUT)

    full = lambda shape: pl.BlockSpec(shape, lambda i: (0, 0))
    out = pl.pallas_call(
        _anfis_block,
        grid=(b // BB,),
        in_specs=[
            pl.BlockSpec((BB, nv), lambda i: (i, 0)),
            full((1, TOTAL_MEM)),
            full((1, TOTAL_MEM)),
            full((1, oc2.shape[1])),
            full((1, N_OUT)),
            full((1, N_OUT)),
            full((8, N_VARS * rpad)),
            full((8, rpad)),
            full((1, TOTAL_MEM)),
        ],
        out_specs=pl.BlockSpec((BB, N_OUT), lambda i: (i, 0)),
        out_shape=jax.ShapeDtypeStruct((b, N_OUT), jnp.float32),
    )(x, c2, s2, oc2, sc2, bi2, rules_cat, orules_t, vm2)
    return out


# speedup vs baseline: 6.9292x; 1.0246x over previous
import jax
import jax.numpy as jnp
from jax.experimental import pallas as pl

N_VARS = 5
TOTAL_MEM = 24
NUM_OC = 18
N_OUT = 2
BB = 1024


def _anfis_block(x_ref, c_ref, s_ref, oc_ref, scale_ref, bias_ref,
                 rules_ref, orules_ref, vm_ref, out_ref):
    rpad = rules_ref.shape[1] // N_VARS
    bb = x_ref.shape[0]
    vm = vm_ref[0, :]
    xv = jnp.zeros((bb, TOTAL_MEM), jnp.float32)
    for v in range(N_VARS):
        xv = jnp.where((vm == v)[None, :], x_ref[:, v][:, None], xv)
    c = c_ref[0, :]
    inv = 0.5 / (s_ref[0, :] ** 2)
    fuzz = jnp.exp(-((xv - c[None, :]) ** 2) * inv[None, :])
    fhi = fuzz.astype(jnp.bfloat16)
    flo = (fuzz - fhi.astype(jnp.float32)).astype(jnp.bfloat16)
    lhs = jnp.concatenate([fhi, flo], axis=1)

    idx = rules_ref[0, :]
    oh = (jax.lax.broadcasted_iota(jnp.int32, (TOTAL_MEM, N_VARS * rpad), 0)
          == idx[None, :]).astype(jnp.bfloat16)
    oh2 = jnp.concatenate([oh, oh], axis=0)
    G = jax.lax.dot(lhs, oh2, preferred_element_type=jnp.float32)

    oc = oc_ref[0, :]
    ows = []
    for j in range(N_OUT):
        orj = orules_ref[j, :]
        owj = jnp.zeros((1, rpad), jnp.float32)
        for k in range(NUM_OC):
            owj = jnp.where((orj == k)[None, :], oc[k], owj)
        ows.append(owj)

    a0 = jnp.zeros((bb, 128), jnp.float32)
    a1 = jnp.zeros((bb, 128), jnp.float32)
    at = jnp.zeros((bb, 128), jnp.float32)
    for kk in range(rpad // 128):
        base = kk * 128
        m = G[:, base:base + 128]
        for v in range(1, N_VARS):
            m = jnp.minimum(m, G[:, v * rpad + base:v * rpad + base + 128])
        a0 = a0 + m * ows[0][:, base:base + 128]
        a1 = a1 + m * ows[1][:, base:base + 128]
        at = at + m
    acc0 = jnp.sum(a0, axis=1, keepdims=True)
    acc1 = jnp.sum(a1, axis=1, keepdims=True)
    total = jnp.sum(at, axis=1, keepdims=True)
    acc = jnp.concatenate([acc0, acc1], axis=1)
    res = jnp.tanh(acc / jnp.maximum(total, 1e-12))
    out_ref[:, :] = res * scale_ref[0, :][None, :] + bias_ref[0, :][None, :]


def kernel(x, centers, sigmas, out_centers, output_scaling, output_bias,
           input_rules, output_rules, var_of_mem):
    b, nv = x.shape
    r = input_rules.shape[0]
    rpad = ((r + 127) // 128) * 128
    rules_cat = jnp.full((N_VARS, rpad), TOTAL_MEM + 7, jnp.int32)
    rules_cat = rules_cat.at[:, :r].set(input_rules.T)
    rules_cat = jnp.broadcast_to(rules_cat.reshape(1, N_VARS * rpad),
                                 (8, N_VARS * rpad))
    orules_t = jnp.full((8, rpad), NUM_OC + 7, jnp.int32)
    orules_t = orules_t.at[:N_OUT, :r].set(output_rules.T)
    c2 = centers.reshape(1, -1)
    s2 = sigmas.reshape(1, -1)
    oc2 = out_centers.reshape(1, -1)
    vm2 = var_of_mem.reshape(1, -1)
    sc2 = output_scaling.reshape(1, N_OUT)
    bi2 = output_bias.reshape(1, N_OUT)

    full = lambda shape: pl.BlockSpec(shape, lambda i: (0, 0))
    out = pl.pallas_call(
        _anfis_block,
        grid=(b // BB,),
        in_specs=[
            pl.BlockSpec((BB, nv), lambda i: (i, 0)),
            full((1, TOTAL_MEM)),
            full((1, TOTAL_MEM)),
            full((1, oc2.shape[1])),
            full((1, N_OUT)),
            full((1, N_OUT)),
            full((8, N_VARS * rpad)),
            full((8, rpad)),
            full((1, TOTAL_MEM)),
        ],
        out_specs=pl.BlockSpec((BB, N_OUT), lambda i: (i, 0)),
        out_shape=jax.ShapeDtypeStruct((b, N_OUT), jnp.float32),
    )(x, c2, s2, oc2, sc2, bi2, rules_cat, orules_t, vm2)
    return out


# speedup vs baseline: 9.0236x; 1.3023x over previous
import jax
import jax.numpy as jnp
from jax.experimental import pallas as pl

N_VARS = 5
TOTAL_MEM = 24
NUM_OC = 18
N_OUT = 2
BB = 1024


def _anfis_block(xt_ref, c_ref, s_ref, oc_ref, scale_ref, bias_ref,
                 rules_ref, orules_ref, vm_ref, out_ref):
    rpad = rules_ref.shape[1] // N_VARS
    bb = xt_ref.shape[1]
    c_t = c_ref[:, :]
    inv_t = 0.5 / (s_ref[:, :] ** 2)
    vm_t = vm_ref[:, :]

    parts = []
    for ch in range(bb // 128):
        xc = xt_ref[:, ch * 128:(ch + 1) * 128]
        xv = jnp.zeros((TOTAL_MEM, 128), jnp.float32)
        for v in range(N_VARS):
            xv = jnp.where(vm_t == v,
                           jnp.broadcast_to(xc[v:v + 1, :], (TOTAL_MEM, 128)),
                           xv)
        f = jnp.exp(-((xv - c_t) ** 2) * inv_t)
        fhi = f.astype(jnp.bfloat16)
        flo = (f - fhi.astype(jnp.float32)).astype(jnp.bfloat16)
        parts.append(jnp.concatenate([fhi, flo], axis=0))
    lhs_t = jnp.concatenate(parts, axis=1)

    idx = rules_ref[0, :]
    oh = (jax.lax.broadcasted_iota(jnp.int32, (TOTAL_MEM, N_VARS * rpad), 0)
          == idx[None, :]).astype(jnp.bfloat16)
    oh2 = jnp.concatenate([oh, oh], axis=0)
    G = jax.lax.dot_general(lhs_t, oh2, (((0,), (0,)), ((), ())),
                            preferred_element_type=jnp.float32)

    oc = oc_ref[0, :]
    ows = []
    for j in range(N_OUT):
        orj = orules_ref[j, :]
        owj = jnp.zeros((1, rpad), jnp.float32)
        for k in range(NUM_OC):
            owj = jnp.where((orj == k)[None, :], oc[k], owj)
        ows.append(owj)

    a0 = jnp.zeros((bb, 128), jnp.float32)
    a1 = jnp.zeros((bb, 128), jnp.float32)
    at = jnp.zeros((bb, 128), jnp.float32)
    for kk in range(rpad // 128):
        base = kk * 128
        m = G[:, base:base + 128]
        for v in range(1, N_VARS):
            m = jnp.minimum(m, G[:, v * rpad + base:v * rpad + base + 128])
        a0 = a0 + m * ows[0][:, base:base + 128]
        a1 = a1 + m * ows[1][:, base:base + 128]
        at = at + m
    acc0 = jnp.sum(a0, axis=1, keepdims=True)
    acc1 = jnp.sum(a1, axis=1, keepdims=True)
    total = jnp.sum(at, axis=1, keepdims=True)
    acc = jnp.concatenate([acc0, acc1], axis=1)
    res = jnp.tanh(acc / jnp.maximum(total, 1e-12))
    out_ref[:, :] = res * scale_ref[0, :][None, :] + bias_ref[0, :][None, :]


def kernel(x, centers, sigmas, out_centers, output_scaling, output_bias,
           input_rules, output_rules, var_of_mem):
    b, nv = x.shape
    r = input_rules.shape[0]
    rpad = ((r + 127) // 128) * 128
    xt = jnp.zeros((8, b), jnp.float32).at[:nv, :].set(x.T)
    rules_cat = jnp.full((N_VARS, rpad), TOTAL_MEM + 7, jnp.int32)
    rules_cat = rules_cat.at[:, :r].set(input_rules.T)
    rules_cat = jnp.broadcast_to(rules_cat.reshape(1, N_VARS * rpad),
                                 (8, N_VARS * rpad))
    orules_t = jnp.full((8, rpad), NUM_OC + 7, jnp.int32)
    orules_t = orules_t.at[:N_OUT, :r].set(output_rules.T)
    c2 = jnp.broadcast_to(centers[:, None], (TOTAL_MEM, 128))
    s2 = jnp.broadcast_to(sigmas[:, None], (TOTAL_MEM, 128))
    vm2 = jnp.broadcast_to(var_of_mem[:, None], (TOTAL_MEM, 128))
    oc2 = out_centers.reshape(1, -1)
    sc2 = output_scaling.reshape(1, N_OUT)
    bi2 = output_bias.reshape(1, N_OUT)

    full = lambda shape: pl.BlockSpec(shape, lambda i: (0, 0))
    out = pl.pallas_call(
        _anfis_block,
        grid=(b // BB,),
        in_specs=[
            pl.BlockSpec((8, BB), lambda i: (0, i)),
            full((TOTAL_MEM, 128)),
            full((TOTAL_MEM, 128)),
            full((1, oc2.shape[1])),
            full((1, N_OUT)),
            full((1, N_OUT)),
            full((8, N_VARS * rpad)),
            full((8, rpad)),
            full((TOTAL_MEM, 128)),
        ],
        out_specs=pl.BlockSpec((BB, N_OUT), lambda i: (i, 0)),
        out_shape=jax.ShapeDtypeStruct((b, N_OUT), jnp.float32),
    )(xt, c2, s2, oc2, sc2, bi2, rules_cat, orules_t, vm2)
    return out
